# baseline (speedup 1.0000x reference)
import functools

import jax
import jax.numpy as jnp
from jax.experimental import pallas as pl
from jax.experimental.pallas import tpu as pltpu

B, T, C = 256, 512, 21
E, K = 4, 2
P, S = 16, 8
PN = (T - P) // S + 1 + 1
D = 128
PRED = 96
DEPTH = 2
KS = 8

R = 96
N_ROWS = B * C

_BN_SCALE = 1.0 / (1.0 + 1e-5) ** 0.5


_INV_SQRT2 = 0.7071067811865476


def _gelu(v):
    return 0.5 * v * (1.0 + jax.lax.erf(v * _INV_SQRT2))


def _expert_body(x_ref, rw_ref, rb_ref, wp_ref, wpb_ref, h0_ref, h0b_ref,
                 h1a_ref, h1ab_ref, h1b_ref, h1bb_ref, dw_ref, dwb_ref,
                 s1_ref, b1_ref, pw0_ref, pwt_ref, pwb_ref, s2_ref, b2_ref,
                 out_ref):
    xv = x_ref[...]
    mean = jnp.mean(xv, axis=1, keepdims=True)
    cen = xv - mean
    var = jnp.mean(cen * cen, axis=1, keepdims=True)
    stdev = jnp.sqrt(var + 1e-5)
    rw = rw_ref[...]
    rb = rb_ref[...]
    xn = cen / stdev * rw + rb

    xp = jnp.concatenate(
        [xn, jnp.broadcast_to(xn[:, T - 1:], (R, S))], axis=1)
    y = xp.reshape(R, PN + 1, S)
    pat = jnp.concatenate([y[:, :PN, :], y[:, 1:, :]], axis=2)

    z = jax.lax.dot_general(pat, wp_ref[...], (((2,), (0,)), ((), ())),
                            preferred_element_type=jnp.float32)
    z = z + wpb_ref[...][None]

    u = jax.lax.dot_general(z.reshape(R, PN * D), h0_ref[...],
                            (((1,), (0,)), ((), ())),
                            preferred_element_type=jnp.float32)
    u = u + h0b_ref[...]

    h = z
    hp = jnp.pad(h, ((0, 0), (0, 0), (3, 4)))
    acc = None
    for k in range(KS):
        term = hp[:, :, k:k + D] * dw_ref[0, :, k][None, :, None]
        acc = term if acc is None else acc + term
    conv = acc + dwb_ref[0][None, :, None]
    r0 = _gelu(conv) * s1_ref[0][None, :, None] + b1_ref[0][None, :, None]
    h = h + r0
    h = jax.lax.dot_general(pw0_ref[...], h, (((1,), (1,)), ((), ())),
                            preferred_element_type=jnp.float32)
    h = h + pwb_ref[0][:, None, None]
    h = _gelu(h) * s2_ref[0][:, None, None] + b2_ref[0][:, None, None]

    hp = jnp.pad(h, ((0, 0), (0, 0), (3, 4)))
    acc = None
    for k in range(KS):
        term = hp[:, :, k:k + D] * dw_ref[1, :, k][:, None, None]
        acc = term if acc is None else acc + term
    conv = acc + dwb_ref[1][:, None, None]
    r1 = _gelu(conv) * s1_ref[1][:, None, None] + b1_ref[1][:, None, None]
    h = h + r1
    h = jax.lax.dot_general(h, pwt_ref[...], (((0,), (0,)), ((), ())),
                            preferred_element_type=jnp.float32)
    h = h + pwb_ref[1][None, None, :]
    h = _gelu(h) * s2_ref[1][None, None, :] + b2_ref[1][None, None, :]

    v1 = jax.lax.dot_general(h.reshape(R, D * PN), h1a_ref[...],
                             (((1,), (0,)), ((), ())),
                             preferred_element_type=jnp.float32)
    v1 = _gelu(v1 + h1ab_ref[...])
    v = jax.lax.dot_general(v1, h1b_ref[...], (((1,), (0,)), ((), ())),
                            preferred_element_type=jnp.float32)
    v = v + h1bb_ref[...]

    o = u + v
    o = (o - rb) / (rw + 1e-10) * stdev + mean
    out_ref[...] = o


def _gating_body(logits_ref, out_ref):
    logits = logits_ref[...]
    ci = jax.lax.broadcasted_iota(jnp.int32, (B, E), 1)
    m1 = jnp.max(logits, axis=1, keepdims=True)
    cand1 = jnp.where(logits == m1, ci, E)
    i1 = jnp.min(cand1, axis=1, keepdims=True)
    oh1 = ci == i1
    l2 = jnp.where(oh1, -jnp.inf, logits)
    m2 = jnp.max(l2, axis=1, keepdims=True)
    cand2 = jnp.where((l2 == m2) & ~oh1, ci, E)
    i2 = jnp.min(cand2, axis=1, keepdims=True)
    oh2 = ci == i2
    e2 = jnp.exp(m2 - m1)
    denom = 1.0 + e2
    g1 = 1.0 / denom
    g2 = e2 / denom
    gates = jnp.where(oh1, g1, 0.0) + jnp.where(oh2, g2, 0.0)

    importance = jnp.sum(gates, axis=0)
    load = jnp.sum((gates > 0.0).astype(jnp.float32), axis=0)

    def cv(v):
        m = jnp.mean(v)
        varv = jnp.sum((v - m) ** 2) / (E - 1)
        return varv / (m * m + 1e-10)

    bal = (cv(importance) + cv(load)) * 0.01
    out_ref[...] = jnp.broadcast_to(bal, (1, 1))


@functools.partial(jax.jit, static_argnums=())
def kernel(x, revin_w, revin_b, start_W, start_b, gate_W, gate_b, WP_W, WP_b,
           h0_W, h0_b, h1a_W, h1a_b, h1b_W, h1b_b, blk_dwW, blk_dwB,
           blk_bn1w, blk_bn1b, blk_pwW, blk_pwB, blk_bn2w, blk_bn2b):
    xt = jnp.transpose(x, (0, 2, 1))
    xr = xt.reshape(N_ROWS, T)

    rw_full = jnp.tile(revin_w, B)[:, None]
    rb_full = jnp.tile(revin_b, B)[:, None]

    wp = WP_W.T
    wpb = WP_b[None, :]
    h0 = h0_W.T
    h0b = h0_b[None, :]
    h1a = jnp.transpose(h1a_W.T.reshape(PN, D, 2 * PRED),
                        (1, 0, 2)).reshape(D * PN, 2 * PRED)
    h1ab = h1a_b[None, :]
    h1b = h1b_W.T
    h1bb = h1b_b[None, :]
    dw = blk_dwW[:, :, 0, :]
    s1 = blk_bn1w * _BN_SCALE
    s2 = blk_bn2w * _BN_SCALE
    pw = blk_pwW[:, :, :, 0]

    n_tiles = N_ROWS // R
    full = lambda *shape: pl.BlockSpec(shape, lambda i: (0,) * len(shape))
    orow = pl.pallas_call(
        _expert_body,
        grid=(n_tiles,),
        in_specs=[
            pl.BlockSpec((R, T), lambda i: (i, 0)),
            pl.BlockSpec((R, 1), lambda i: (i, 0)),
            pl.BlockSpec((R, 1), lambda i: (i, 0)),
            full(P, D), full(1, D),
            full(PN * D, PRED), full(1, PRED),
            full(D * PN, 2 * PRED), full(1, 2 * PRED),
            full(2 * PRED, PRED), full(1, PRED),
            full(DEPTH, PN, KS), full(DEPTH, PN),
            full(DEPTH, PN), full(DEPTH, PN),
            full(PN, PN), full(PN, PN), full(DEPTH, PN),
            full(DEPTH, PN), full(DEPTH, PN),
        ],
        out_specs=pl.BlockSpec((R, PRED), lambda i: (i, 0)),
        out_shape=jax.ShapeDtypeStruct((N_ROWS, PRED), jnp.float32),
        compiler_params=pltpu.CompilerParams(
            dimension_semantics=("arbitrary",)),
    )(xr, rw_full, rb_full, wp, wpb, h0, h0b, h1a, h1ab, h1b, h1bb,
      dw, blk_dwB, s1, blk_bn1b, pw[0], pw[1].T, blk_pwB, s2, blk_bn2b)

    out = jnp.transpose(orow.reshape(B, C, PRED), (0, 2, 1))

    mean_g = jax.lax.stop_gradient(jnp.mean(x, axis=1, keepdims=True))
    stdev_g = jax.lax.stop_gradient(
        jnp.sqrt(jnp.var(x, axis=1, keepdims=True) + 1e-5))
    xn_g = (x - mean_g) / stdev_g * revin_w + revin_b
    xg = jnp.squeeze(xn_g @ start_W.T + start_b, -1)
    logits = xg @ gate_W.T + gate_b

    balance = pl.pallas_call(
        _gating_body,
        in_specs=[pl.BlockSpec((B, E), lambda: (0, 0))],
        out_specs=pl.BlockSpec((1, 1), lambda: (0, 0)),
        out_shape=jax.ShapeDtypeStruct((1, 1), jnp.float32),
    )(logits)

    return (out, balance[0, 0])


# speedup vs baseline: 1.2964x; 1.2964x over previous
import functools

import jax
import jax.numpy as jnp
from jax.experimental import pallas as pl
from jax.experimental.pallas import tpu as pltpu

B, T, C = 256, 512, 21
E, K = 4, 2
P, S = 16, 8
PN = (T - P) // S + 1 + 1
D = 128
PRED = 96
DEPTH = 2
KS = 8

R = 96
CHUNK = 64
N_ROWS = B * C

_BN_SCALE = 1.0 / (1.0 + 1e-5) ** 0.5


_INV_SQRT2 = 0.7071067811865476


def _gelu(v):
    return 0.5 * v * (1.0 + jax.lax.erf(v * _INV_SQRT2))


def _expert_body(x_ref, rw_ref, rb_ref, wp_ref, wpb_ref, h0_ref, h0b_ref,
                 h1a_ref, h1ab_ref, h1b_ref, h1bb_ref, dw_ref, dwb_ref,
                 s1_ref, b1_ref, pw0_ref, pwt_ref, pwb_ref, s2_ref, b2_ref,
                 out_ref):
    xv = x_ref[...]
    mean = jnp.mean(xv, axis=1, keepdims=True)
    cen = xv - mean
    var = jnp.mean(cen * cen, axis=1, keepdims=True)
    stdev = jnp.sqrt(var + 1e-5)
    rw = rw_ref[...]
    rb = rb_ref[...]
    xn = cen / stdev * rw + rb

    xp = jnp.concatenate(
        [xn, jnp.broadcast_to(xn[:, T - 1:], (R, S))], axis=1)
    y = xp.reshape(R, PN + 1, S)
    pat = jnp.concatenate([y[:, :PN, :], y[:, 1:, :]], axis=2)

    z = jax.lax.dot_general(pat, wp_ref[...], (((2,), (0,)), ((), ())),
                            preferred_element_type=jnp.float32)
    z = z + wpb_ref[...][None]

    u = h0b_ref[...]
    for g in range(0, PN, CHUNK):
        ug = jax.lax.dot_general(
            z[:, g:g + CHUNK, :], h0_ref[g:g + CHUNK],
            (((2,), (1,)), ((1,), (0,))),
            preferred_element_type=jnp.float32)
        u = u + jnp.sum(ug, axis=0)

    h = z
    hp = jnp.pad(h, ((0, 0), (0, 0), (3, 4)))
    acc = None
    for k in range(KS):
        term = hp[:, :, k:k + D] * dw_ref[0, :, k][None, :, None]
        acc = term if acc is None else acc + term
    conv = acc + dwb_ref[0][None, :, None]
    r0 = _gelu(conv) * s1_ref[0][None, :, None] + b1_ref[0][None, :, None]
    h = h + r0
    h = jax.lax.dot_general(pw0_ref[...], h, (((1,), (1,)), ((), ())),
                            preferred_element_type=jnp.float32)
    h = h + pwb_ref[0][:, None, None]
    h = _gelu(h) * s2_ref[0][:, None, None] + b2_ref[0][:, None, None]

    hp = jnp.pad(h, ((0, 0), (0, 0), (3, 4)))
    acc = None
    for k in range(KS):
        term = hp[:, :, k:k + D] * dw_ref[1, :, k][:, None, None]
        acc = term if acc is None else acc + term
    conv = acc + dwb_ref[1][:, None, None]
    r1 = _gelu(conv) * s1_ref[1][:, None, None] + b1_ref[1][:, None, None]
    h = h + r1
    h = jax.lax.dot_general(pwt_ref[...], h, (((1,), (0,)), ((), ())),
                            preferred_element_type=jnp.float32)
    h = h + pwb_ref[1][:, None, None]
    h = _gelu(h) * s2_ref[1][:, None, None] + b2_ref[1][:, None, None]

    v1 = h1ab_ref[...]
    for g in range(0, PN, CHUNK):
        vg = jax.lax.dot_general(
            h[g:g + CHUNK], h1a_ref[g:g + CHUNK],
            (((2,), (1,)), ((0,), (0,))),
            preferred_element_type=jnp.float32)
        v1 = v1 + jnp.sum(vg, axis=0)
    v1 = _gelu(v1)
    v = jax.lax.dot_general(v1, h1b_ref[...], (((1,), (0,)), ((), ())),
                            preferred_element_type=jnp.float32)
    v = v + h1bb_ref[...]

    o = u + v
    o = (o - rb) / (rw + 1e-10) * stdev + mean
    out_ref[...] = o


def _gating_body(logits_ref, out_ref):
    logits = logits_ref[...]
    ci = jax.lax.broadcasted_iota(jnp.int32, (B, E), 1)
    m1 = jnp.max(logits, axis=1, keepdims=True)
    cand1 = jnp.where(logits == m1, ci, E)
    i1 = jnp.min(cand1, axis=1, keepdims=True)
    oh1 = ci == i1
    l2 = jnp.where(oh1, -jnp.inf, logits)
    m2 = jnp.max(l2, axis=1, keepdims=True)
    cand2 = jnp.where((l2 == m2) & ~oh1, ci, E)
    i2 = jnp.min(cand2, axis=1, keepdims=True)
    oh2 = ci == i2
    e2 = jnp.exp(m2 - m1)
    denom = 1.0 + e2
    g1 = 1.0 / denom
    g2 = e2 / denom
    gates = jnp.where(oh1, g1, 0.0) + jnp.where(oh2, g2, 0.0)

    importance = jnp.sum(gates, axis=0)
    load = jnp.sum((gates > 0.0).astype(jnp.float32), axis=0)

    def cv(v):
        m = jnp.mean(v)
        varv = jnp.sum((v - m) ** 2) / (E - 1)
        return varv / (m * m + 1e-10)

    bal = (cv(importance) + cv(load)) * 0.01
    out_ref[...] = jnp.broadcast_to(bal, (1, 1))


@functools.partial(jax.jit, static_argnums=())
def kernel(x, revin_w, revin_b, start_W, start_b, gate_W, gate_b, WP_W, WP_b,
           h0_W, h0_b, h1a_W, h1a_b, h1b_W, h1b_b, blk_dwW, blk_dwB,
           blk_bn1w, blk_bn1b, blk_pwW, blk_pwB, blk_bn2w, blk_bn2b):
    xt = jnp.transpose(x, (0, 2, 1))
    xr = xt.reshape(N_ROWS, T)

    rw_full = jnp.tile(revin_w, B)[:, None]
    rb_full = jnp.tile(revin_b, B)[:, None]

    wp = WP_W.T
    wpb = WP_b[None, :]
    h0 = h0_W.T.reshape(PN, D, PRED)
    h0b = h0_b[None, :]
    h1a = h1a_W.T.reshape(PN, D, 2 * PRED)
    h1ab = h1a_b[None, :]
    h1b = h1b_W.T
    h1bb = h1b_b[None, :]
    dw = blk_dwW[:, :, 0, :]
    s1 = blk_bn1w * _BN_SCALE
    s2 = blk_bn2w * _BN_SCALE
    pw = blk_pwW[:, :, :, 0]

    n_tiles = N_ROWS // R
    full = lambda *shape: pl.BlockSpec(shape, lambda i: (0,) * len(shape))
    orow = pl.pallas_call(
        _expert_body,
        grid=(n_tiles,),
        in_specs=[
            pl.BlockSpec((R, T), lambda i: (i, 0)),
            pl.BlockSpec((R, 1), lambda i: (i, 0)),
            pl.BlockSpec((R, 1), lambda i: (i, 0)),
            full(P, D), full(1, D),
            full(PN, D, PRED), full(1, PRED),
            full(PN, D, 2 * PRED), full(1, 2 * PRED),
            full(2 * PRED, PRED), full(1, PRED),
            full(DEPTH, PN, KS), full(DEPTH, PN),
            full(DEPTH, PN), full(DEPTH, PN),
            full(PN, PN), full(PN, PN), full(DEPTH, PN),
            full(DEPTH, PN), full(DEPTH, PN),
        ],
        out_specs=pl.BlockSpec((R, PRED), lambda i: (i, 0)),
        out_shape=jax.ShapeDtypeStruct((N_ROWS, PRED), jnp.float32),
        compiler_params=pltpu.CompilerParams(
            dimension_semantics=("arbitrary",)),
    )(xr, rw_full, rb_full, wp, wpb, h0, h0b, h1a, h1ab, h1b, h1bb,
      dw, blk_dwB, s1, blk_bn1b, pw[0], pw[1], blk_pwB, s2, blk_bn2b)

    out = jnp.transpose(orow.reshape(B, C, PRED), (0, 2, 1))

    mean_g = jax.lax.stop_gradient(jnp.mean(x, axis=1, keepdims=True))
    stdev_g = jax.lax.stop_gradient(
        jnp.sqrt(jnp.var(x, axis=1, keepdims=True) + 1e-5))
    xn_g = (x - mean_g) / stdev_g * revin_w + revin_b
    xg = jnp.squeeze(xn_g @ start_W.T + start_b, -1)
    logits = xg @ gate_W.T + gate_b

    balance = pl.pallas_call(
        _gating_body,
        in_specs=[pl.BlockSpec((B, E), lambda: (0, 0))],
        out_specs=pl.BlockSpec((1, 1), lambda: (0, 0)),
        out_shape=jax.ShapeDtypeStruct((1, 1), jnp.float32),
    )(logits)

    return (out, balance[0, 0])


# speedup vs baseline: 1.2967x; 1.0003x over previous
import functools

import jax
import jax.numpy as jnp
from jax.experimental import pallas as pl
from jax.experimental.pallas import tpu as pltpu

B, T, C = 256, 512, 21
E, K = 4, 2
P, S = 16, 8
PN = (T - P) // S + 1 + 1
D = 128
PRED = 96
DEPTH = 2
KS = 8

R = 96
CHUNK = 16
N_ROWS = B * C

_BN_SCALE = 1.0 / (1.0 + 1e-5) ** 0.5


_INV_SQRT2 = 0.7071067811865476


def _gelu(v):
    return 0.5 * v * (1.0 + jax.lax.erf(v * _INV_SQRT2))


def _expert_body(x_ref, rw_ref, rb_ref, wp_ref, wpb_ref, h0_ref, h0b_ref,
                 h1a_ref, h1ab_ref, h1b_ref, h1bb_ref, dw_ref, dwb_ref,
                 s1_ref, b1_ref, pw0_ref, pwt_ref, pwb_ref, s2_ref, b2_ref,
                 out_ref):
    xv = x_ref[...]
    mean = jnp.mean(xv, axis=1, keepdims=True)
    cen = xv - mean
    var = jnp.mean(cen * cen, axis=1, keepdims=True)
    stdev = jnp.sqrt(var + 1e-5)
    rw = rw_ref[...]
    rb = rb_ref[...]
    xn = cen / stdev * rw + rb

    xp = jnp.concatenate(
        [xn, jnp.broadcast_to(xn[:, T - 1:], (R, S))], axis=1)
    y = xp.reshape(R, PN + 1, S)
    pat = jnp.concatenate([y[:, :PN, :], y[:, 1:, :]], axis=2)

    z = jax.lax.dot_general(pat, wp_ref[...], (((2,), (0,)), ((), ())),
                            preferred_element_type=jnp.float32)
    z = z + wpb_ref[...][None]

    u = h0b_ref[...]
    for g in range(0, PN, CHUNK):
        ug = jax.lax.dot_general(
            z[:, g:g + CHUNK, :], h0_ref[g:g + CHUNK],
            (((2,), (1,)), ((1,), (0,))),
            preferred_element_type=jnp.float32)
        u = u + jnp.sum(ug, axis=0)

    h = z
    hp = jnp.pad(h, ((0, 0), (0, 0), (3, 4)))
    acc = None
    for k in range(KS):
        term = hp[:, :, k:k + D] * dw_ref[0, :, k][None, :, None]
        acc = term if acc is None else acc + term
    conv = acc + dwb_ref[0][None, :, None]
    r0 = _gelu(conv) * s1_ref[0][None, :, None] + b1_ref[0][None, :, None]
    h = h + r0
    h = jax.lax.dot_general(pw0_ref[...], h, (((1,), (1,)), ((), ())),
                            preferred_element_type=jnp.float32)
    h = h + pwb_ref[0][:, None, None]
    h = _gelu(h) * s2_ref[0][:, None, None] + b2_ref[0][:, None, None]

    hp = jnp.pad(h, ((0, 0), (0, 0), (3, 4)))
    acc = None
    for k in range(KS):
        term = hp[:, :, k:k + D] * dw_ref[1, :, k][:, None, None]
        acc = term if acc is None else acc + term
    conv = acc + dwb_ref[1][:, None, None]
    r1 = _gelu(conv) * s1_ref[1][:, None, None] + b1_ref[1][:, None, None]
    h = h + r1
    h = jax.lax.dot_general(pwt_ref[...], h, (((1,), (0,)), ((), ())),
                            preferred_element_type=jnp.float32)
    h = h + pwb_ref[1][:, None, None]
    h = _gelu(h) * s2_ref[1][:, None, None] + b2_ref[1][:, None, None]

    v1 = h1ab_ref[...]
    for g in range(0, PN, CHUNK):
        vg = jax.lax.dot_general(
            h[g:g + CHUNK], h1a_ref[g:g + CHUNK],
            (((2,), (1,)), ((0,), (0,))),
            preferred_element_type=jnp.float32)
        v1 = v1 + jnp.sum(vg, axis=0)
    v1 = _gelu(v1)
    v = jax.lax.dot_general(v1, h1b_ref[...], (((1,), (0,)), ((), ())),
                            preferred_element_type=jnp.float32)
    v = v + h1bb_ref[...]

    o = u + v
    o = (o - rb) / (rw + 1e-10) * stdev + mean
    out_ref[...] = o


def _gating_body(logits_ref, out_ref):
    logits = logits_ref[...]
    ci = jax.lax.broadcasted_iota(jnp.int32, (B, E), 1)
    m1 = jnp.max(logits, axis=1, keepdims=True)
    cand1 = jnp.where(logits == m1, ci, E)
    i1 = jnp.min(cand1, axis=1, keepdims=True)
    oh1 = ci == i1
    l2 = jnp.where(oh1, -jnp.inf, logits)
    m2 = jnp.max(l2, axis=1, keepdims=True)
    cand2 = jnp.where((l2 == m2) & ~oh1, ci, E)
    i2 = jnp.min(cand2, axis=1, keepdims=True)
    oh2 = ci == i2
    e2 = jnp.exp(m2 - m1)
    denom = 1.0 + e2
    g1 = 1.0 / denom
    g2 = e2 / denom
    gates = jnp.where(oh1, g1, 0.0) + jnp.where(oh2, g2, 0.0)

    importance = jnp.sum(gates, axis=0)
    load = jnp.sum((gates > 0.0).astype(jnp.float32), axis=0)

    def cv(v):
        m = jnp.mean(v)
        varv = jnp.sum((v - m) ** 2) / (E - 1)
        return varv / (m * m + 1e-10)

    bal = (cv(importance) + cv(load)) * 0.01
    out_ref[...] = jnp.broadcast_to(bal, (1, 1))


@functools.partial(jax.jit, static_argnums=())
def kernel(x, revin_w, revin_b, start_W, start_b, gate_W, gate_b, WP_W, WP_b,
           h0_W, h0_b, h1a_W, h1a_b, h1b_W, h1b_b, blk_dwW, blk_dwB,
           blk_bn1w, blk_bn1b, blk_pwW, blk_pwB, blk_bn2w, blk_bn2b):
    xt = jnp.transpose(x, (0, 2, 1))
    xr = xt.reshape(N_ROWS, T)

    rw_full = jnp.tile(revin_w, B)[:, None]
    rb_full = jnp.tile(revin_b, B)[:, None]

    wp = WP_W.T
    wpb = WP_b[None, :]
    h0 = h0_W.T.reshape(PN, D, PRED)
    h0b = h0_b[None, :]
    h1a = h1a_W.T.reshape(PN, D, 2 * PRED)
    h1ab = h1a_b[None, :]
    h1b = h1b_W.T
    h1bb = h1b_b[None, :]
    dw = blk_dwW[:, :, 0, :]
    s1 = blk_bn1w * _BN_SCALE
    s2 = blk_bn2w * _BN_SCALE
    pw = blk_pwW[:, :, :, 0]

    n_tiles = N_ROWS // R
    full = lambda *shape: pl.BlockSpec(shape, lambda i: (0,) * len(shape))
    orow = pl.pallas_call(
        _expert_body,
        grid=(n_tiles,),
        in_specs=[
            pl.BlockSpec((R, T), lambda i: (i, 0)),
            pl.BlockSpec((R, 1), lambda i: (i, 0)),
            pl.BlockSpec((R, 1), lambda i: (i, 0)),
            full(P, D), full(1, D),
            full(PN, D, PRED), full(1, PRED),
            full(PN, D, 2 * PRED), full(1, 2 * PRED),
            full(2 * PRED, PRED), full(1, PRED),
            full(DEPTH, PN, KS), full(DEPTH, PN),
            full(DEPTH, PN), full(DEPTH, PN),
            full(PN, PN), full(PN, PN), full(DEPTH, PN),
            full(DEPTH, PN), full(DEPTH, PN),
        ],
        out_specs=pl.BlockSpec((R, PRED), lambda i: (i, 0)),
        out_shape=jax.ShapeDtypeStruct((N_ROWS, PRED), jnp.float32),
        compiler_params=pltpu.CompilerParams(
            dimension_semantics=("arbitrary",)),
    )(xr, rw_full, rb_full, wp, wpb, h0, h0b, h1a, h1ab, h1b, h1bb,
      dw, blk_dwB, s1, blk_bn1b, pw[0], pw[1], blk_pwB, s2, blk_bn2b)

    out = jnp.transpose(orow.reshape(B, C, PRED), (0, 2, 1))

    mean_g = jax.lax.stop_gradient(jnp.mean(x, axis=1, keepdims=True))
    stdev_g = jax.lax.stop_gradient(
        jnp.sqrt(jnp.var(x, axis=1, keepdims=True) + 1e-5))
    xn_g = (x - mean_g) / stdev_g * revin_w + revin_b
    xg = jnp.squeeze(xn_g @ start_W.T + start_b, -1)
    logits = xg @ gate_W.T + gate_b

    balance = pl.pallas_call(
        _gating_body,
        in_specs=[pl.BlockSpec((B, E), lambda: (0, 0))],
        out_specs=pl.BlockSpec((1, 1), lambda: (0, 0)),
        out_shape=jax.ShapeDtypeStruct((1, 1), jnp.float32),
    )(logits)

    return (out, balance[0, 0])


# speedup vs baseline: 1.3197x; 1.0177x over previous
import functools

import jax
import jax.numpy as jnp
from jax.experimental import pallas as pl
from jax.experimental.pallas import tpu as pltpu

B, T, C = 256, 512, 21
E, K = 4, 2
P, S = 16, 8
PN = (T - P) // S + 1 + 1
D = 128
PRED = 96
DEPTH = 2
KS = 8

R = 96
CHUNK = 16
N_ROWS = B * C

_BN_SCALE = 1.0 / (1.0 + 1e-5) ** 0.5


_INV_SQRT2 = 0.7071067811865476


def _gelu(v):
    return 0.5 * v * (1.0 + jax.lax.erf(v * _INV_SQRT2))


def _expert_body(x_ref, rw_ref, rb_ref, wp_ref, wpb_ref, h0_ref, h0b_ref,
                 h1a_ref, h1ab_ref, h1b_ref, h1bb_ref, dw_ref, dwb_ref,
                 s1_ref, b1_ref, pw0_ref, pwt_ref, pwb_ref, s2_ref, b2_ref,
                 out_ref):
    xv = x_ref[...]
    mean = jnp.mean(xv, axis=1, keepdims=True)
    cen = xv - mean
    var = jnp.mean(cen * cen, axis=1, keepdims=True)
    stdev = jnp.sqrt(var + 1e-5)
    rw = rw_ref[...]
    rb = rb_ref[...]
    xn = cen / stdev * rw + rb

    xp = jnp.concatenate(
        [xn, jnp.broadcast_to(xn[:, T - 1:], (R, S))], axis=1)
    y = xp.reshape(R, PN + 1, S)
    pat = jnp.concatenate([y[:, :PN, :], y[:, 1:, :]], axis=2)

    z = jax.lax.dot_general(pat.astype(jnp.bfloat16), wp_ref[...],
                            (((2,), (0,)), ((), ())),
                            preferred_element_type=jnp.float32)
    z = z + wpb_ref[...][None]

    u = h0b_ref[...]
    z_bf = z.astype(jnp.bfloat16)
    for g in range(0, PN, CHUNK):
        ug = jax.lax.dot_general(
            z_bf[:, g:g + CHUNK, :], h0_ref[g:g + CHUNK],
            (((2,), (1,)), ((1,), (0,))),
            preferred_element_type=jnp.float32)
        u = u + jnp.sum(ug, axis=0)

    h = z
    hp = jnp.pad(h, ((0, 0), (0, 0), (3, 4)))
    acc = None
    for k in range(KS):
        term = hp[:, :, k:k + D] * dw_ref[0, :, k][None, :, None]
        acc = term if acc is None else acc + term
    conv = acc + dwb_ref[0][None, :, None]
    r0 = _gelu(conv) * s1_ref[0][None, :, None] + b1_ref[0][None, :, None]
    h = h + r0
    h = jax.lax.dot_general(pw0_ref[...], h.astype(jnp.bfloat16),
                            (((1,), (1,)), ((), ())),
                            preferred_element_type=jnp.float32)
    h = h + pwb_ref[0][:, None, None]
    h = _gelu(h) * s2_ref[0][:, None, None] + b2_ref[0][:, None, None]

    hp = jnp.pad(h, ((0, 0), (0, 0), (3, 4)))
    acc = None
    for k in range(KS):
        term = hp[:, :, k:k + D] * dw_ref[1, :, k][:, None, None]
        acc = term if acc is None else acc + term
    conv = acc + dwb_ref[1][:, None, None]
    r1 = _gelu(conv) * s1_ref[1][:, None, None] + b1_ref[1][:, None, None]
    h = h + r1
    h = jax.lax.dot_general(pwt_ref[...], h.astype(jnp.bfloat16),
                            (((1,), (0,)), ((), ())),
                            preferred_element_type=jnp.float32)
    h = h + pwb_ref[1][:, None, None]
    h = _gelu(h) * s2_ref[1][:, None, None] + b2_ref[1][:, None, None]

    v1 = h1ab_ref[...]
    h_bf = h.astype(jnp.bfloat16)
    for g in range(0, PN, CHUNK):
        vg = jax.lax.dot_general(
            h_bf[g:g + CHUNK], h1a_ref[g:g + CHUNK],
            (((2,), (1,)), ((0,), (0,))),
            preferred_element_type=jnp.float32)
        v1 = v1 + jnp.sum(vg, axis=0)
    v1 = _gelu(v1)
    v = jax.lax.dot_general(v1.astype(jnp.bfloat16), h1b_ref[...],
                            (((1,), (0,)), ((), ())),
                            preferred_element_type=jnp.float32)
    v = v + h1bb_ref[...]

    o = u + v
    o = (o - rb) / (rw + 1e-10) * stdev + mean
    out_ref[...] = o


def _gating_body(logits_ref, out_ref):
    logits = logits_ref[...]
    ci = jax.lax.broadcasted_iota(jnp.int32, (B, E), 1)
    m1 = jnp.max(logits, axis=1, keepdims=True)
    cand1 = jnp.where(logits == m1, ci, E)
    i1 = jnp.min(cand1, axis=1, keepdims=True)
    oh1 = ci == i1
    l2 = jnp.where(oh1, -jnp.inf, logits)
    m2 = jnp.max(l2, axis=1, keepdims=True)
    cand2 = jnp.where((l2 == m2) & ~oh1, ci, E)
    i2 = jnp.min(cand2, axis=1, keepdims=True)
    oh2 = ci == i2
    e2 = jnp.exp(m2 - m1)
    denom = 1.0 + e2
    g1 = 1.0 / denom
    g2 = e2 / denom
    gates = jnp.where(oh1, g1, 0.0) + jnp.where(oh2, g2, 0.0)

    importance = jnp.sum(gates, axis=0)
    load = jnp.sum((gates > 0.0).astype(jnp.float32), axis=0)

    def cv(v):
        m = jnp.mean(v)
        varv = jnp.sum((v - m) ** 2) / (E - 1)
        return varv / (m * m + 1e-10)

    bal = (cv(importance) + cv(load)) * 0.01
    out_ref[...] = jnp.broadcast_to(bal, (1, 1))


@functools.partial(jax.jit, static_argnums=())
def kernel(x, revin_w, revin_b, start_W, start_b, gate_W, gate_b, WP_W, WP_b,
           h0_W, h0_b, h1a_W, h1a_b, h1b_W, h1b_b, blk_dwW, blk_dwB,
           blk_bn1w, blk_bn1b, blk_pwW, blk_pwB, blk_bn2w, blk_bn2b):
    xt = jnp.transpose(x, (0, 2, 1))
    xr = xt.reshape(N_ROWS, T)

    rw_full = jnp.tile(revin_w, B)[:, None]
    rb_full = jnp.tile(revin_b, B)[:, None]

    wp = WP_W.T.astype(jnp.bfloat16)
    wpb = WP_b[None, :]
    h0 = h0_W.T.reshape(PN, D, PRED).astype(jnp.bfloat16)
    h0b = h0_b[None, :]
    h1a = h1a_W.T.reshape(PN, D, 2 * PRED).astype(jnp.bfloat16)
    h1ab = h1a_b[None, :]
    h1b = h1b_W.T.astype(jnp.bfloat16)
    h1bb = h1b_b[None, :]
    dw = blk_dwW[:, :, 0, :]
    s1 = blk_bn1w * _BN_SCALE
    s2 = blk_bn2w * _BN_SCALE
    pw = blk_pwW[:, :, :, 0].astype(jnp.bfloat16)

    n_tiles = N_ROWS // R
    full = lambda *shape: pl.BlockSpec(shape, lambda i: (0,) * len(shape))
    orow = pl.pallas_call(
        _expert_body,
        grid=(n_tiles,),
        in_specs=[
            pl.BlockSpec((R, T), lambda i: (i, 0)),
            pl.BlockSpec((R, 1), lambda i: (i, 0)),
            pl.BlockSpec((R, 1), lambda i: (i, 0)),
            full(P, D), full(1, D),
            full(PN, D, PRED), full(1, PRED),
            full(PN, D, 2 * PRED), full(1, 2 * PRED),
            full(2 * PRED, PRED), full(1, PRED),
            full(DEPTH, PN, KS), full(DEPTH, PN),
            full(DEPTH, PN), full(DEPTH, PN),
            full(PN, PN), full(PN, PN), full(DEPTH, PN),
            full(DEPTH, PN), full(DEPTH, PN),
        ],
        out_specs=pl.BlockSpec((R, PRED), lambda i: (i, 0)),
        out_shape=jax.ShapeDtypeStruct((N_ROWS, PRED), jnp.float32),
        compiler_params=pltpu.CompilerParams(
            dimension_semantics=("arbitrary",)),
    )(xr, rw_full, rb_full, wp, wpb, h0, h0b, h1a, h1ab, h1b, h1bb,
      dw, blk_dwB, s1, blk_bn1b, pw[0], pw[1], blk_pwB, s2, blk_bn2b)

    out = jnp.transpose(orow.reshape(B, C, PRED), (0, 2, 1))

    mean_g = jax.lax.stop_gradient(jnp.mean(x, axis=1, keepdims=True))
    stdev_g = jax.lax.stop_gradient(
        jnp.sqrt(jnp.var(x, axis=1, keepdims=True) + 1e-5))
    xn_g = (x - mean_g) / stdev_g * revin_w + revin_b
    xg = jnp.squeeze(xn_g @ start_W.T + start_b, -1)
    logits = xg @ gate_W.T + gate_b

    balance = pl.pallas_call(
        _gating_body,
        in_specs=[pl.BlockSpec((B, E), lambda: (0, 0))],
        out_specs=pl.BlockSpec((1, 1), lambda: (0, 0)),
        out_shape=jax.ShapeDtypeStruct((1, 1), jnp.float32),
    )(logits)

    return (out, balance[0, 0])
